# Initial kernel scaffold; baseline (speedup 1.0000x reference)
import jax
import jax.numpy as jnp
from jax.experimental import pallas as pl


def kernel(x, edge_index, parent_mask, W_in, b_in, mW0, mb0, uW0, ub0, mW1, mb1, uW1, ub1, W_out, b_out):
    raise NotImplementedError("write your pallas kernel here")


# speedup vs baseline: 7.4705x; 7.4705x over previous
import functools

import jax
import jax.numpy as jnp
from jax import lax
from jax.experimental import pallas as pl
from jax.experimental.pallas import tpu as pltpu
from jax.experimental.pallas import tpu_sc as plsc

N = 10000
E = 320000
IN_DIM = 128
HID = 32
EMB = 32
R3 = 3 * N
TROWS = R3 + 8
NW = 32
B = 128
CH = 236
CC = 80
ARM = 30080
ARC = 10112
TRASHM = 30000
TRASHC = 10000

_SC_PARAMS = pltpu.CompilerParams(use_tc_tiling_on_sc=False)


def _make_sc_main(ch, ar):
    rpt = ar // 16
    _MESH = plsc.VectorSubcoreMesh(core_axis_name="c", subcore_axis_name="s")

    @functools.partial(
        pl.kernel,
        out_type=jax.ShapeDtypeStruct((2, ar, HID), jnp.float32),
        mesh=_MESH,
        scratch_types=[
            pltpu.VMEM((ch, B), jnp.int32),
            pltpu.VMEM((ch, B), jnp.int32),
            pltpu.VMEM((B, HID), jnp.float32),
            pltpu.VMEM_SHARED((ar, HID), jnp.float32),
            pltpu.SemaphoreType.DMA,
        ],
        compiler_params=_SC_PARAMS,
    )
    def sc_main(t_hbm, g_hbm, s_hbm, z_hbm, out_hbm, gidx, sidx, rows, acc, sem):
        c = lax.axis_index("c")
        s = lax.axis_index("s")
        w = c * 16 + s

        pltpu.sync_copy(z_hbm, acc.at[pl.ds(s * rpt, rpt)])
        pltpu.sync_copy(g_hbm.at[w], gidx)
        pltpu.sync_copy(s_hbm.at[w], sidx)
        plsc.subcore_barrier()

        def body(j, carry):
            pltpu.async_copy(t_hbm.at[gidx.at[j]], rows, sem).wait()
            pltpu.sync_copy(rows, acc.at[sidx.at[j]], add=True)
            return carry

        lax.fori_loop(0, ch, body, 0)
        plsc.subcore_barrier()

        pltpu.sync_copy(acc.at[pl.ds(s * rpt, rpt)],
                        out_hbm.at[c, pl.ds(s * rpt, rpt)])

    return sc_main


def _make_sc_counts(cc, ar):
    rpt = ar // 16
    _MESH = plsc.VectorSubcoreMesh(core_axis_name="c", subcore_axis_name="s")

    @functools.partial(
        pl.kernel,
        out_type=jax.ShapeDtypeStruct((2, ar, HID), jnp.float32),
        mesh=_MESH,
        scratch_types=[
            pltpu.VMEM((cc, B), jnp.int32),
            pltpu.VMEM((B, HID), jnp.float32),
            pltpu.VMEM_SHARED((ar, HID), jnp.float32),
        ],
        compiler_params=_SC_PARAMS,
    )
    def sc_counts(ones_hbm, c_hbm, z_hbm, out_hbm, cidx, rows, acc):
        c = lax.axis_index("c")
        s = lax.axis_index("s")
        w = c * 16 + s

        pltpu.sync_copy(z_hbm, acc.at[pl.ds(s * rpt, rpt)])
        pltpu.sync_copy(c_hbm.at[w], cidx)
        pltpu.sync_copy(ones_hbm, rows)
        plsc.subcore_barrier()

        def body(j, carry):
            pltpu.sync_copy(rows, acc.at[cidx.at[j]], add=True)
            return carry

        lax.fori_loop(0, cc, body, 0)
        plsc.subcore_barrier()
        pltpu.sync_copy(acc.at[pl.ds(s * rpt, rpt)],
                        out_hbm.at[c, pl.ds(s * rpt, rpt)])

    return sc_counts


_sc_main = functools.cache(lambda: _make_sc_main(CH, ARM))
_sc_counts = functools.cache(lambda: _make_sc_counts(CC, ARC))


RB = 1000


def _rb(a):
    return a.astype(jnp.bfloat16).astype(jnp.float32)


def _dot(a, b):
    return jnp.dot(_rb(a), _rb(b), preferred_element_type=jnp.float32,
                   precision=jax.lax.Precision.HIGHEST)


def _tc_in_body(x_ref, w_ref, b_ref, o_ref):
    o_ref[...] = _rb(jnp.maximum(_dot(x_ref[...], w_ref[...]) + b_ref[...], 0.0))


_tc_in = pl.pallas_call(
    _tc_in_body,
    grid=(R3 // RB,),
    in_specs=[
        pl.BlockSpec((RB, IN_DIM), lambda i: (i, 0)),
        pl.BlockSpec((IN_DIM, HID), lambda i: (0, 0)),
        pl.BlockSpec((1, HID), lambda i: (0, 0)),
    ],
    out_specs=pl.BlockSpec((RB, HID), lambda i: (i, 0)),
    out_shape=jax.ShapeDtypeStruct((R3, HID), jnp.float32),
)


def _tc_update_body(h_ref, p_ref, pc_ref, mw_ref, mb_ref, uw_ref, ub_ref,
                    ho_ref):
    h = h_ref[...]
    S = p_ref[0] + p_ref[1]
    cnt = pc_ref[0, :, 0:1] + pc_ref[1, :, 0:1]
    mw = mw_ref[...]
    uw = uw_ref[...]
    hw = _dot(h, mw[:HID])
    sw = jnp.dot(S, _rb(mw[HID:]), preferred_element_type=jnp.float32,
                 precision=jax.lax.Precision.HIGHEST)
    summed = hw * cnt + sw + cnt * mb_ref[...]
    aggr = summed / jnp.maximum(cnt, 1.0)
    ho_ref[...] = _rb(jnp.maximum(
        _dot(h, uw[:HID]) + _dot(aggr, uw[HID:]) + ub_ref[...], 0.0))


_tc_update = pl.pallas_call(
    _tc_update_body,
    grid=(R3 // RB,),
    in_specs=[
        pl.BlockSpec((RB, HID), lambda i: (i, 0)),
        pl.BlockSpec((2, RB, HID), lambda i: (0, i, 0)),
        pl.BlockSpec((2, RB, HID), lambda i: (0, i % (N // RB), 0)),
        pl.BlockSpec((2 * HID, HID), lambda i: (0, 0)),
        pl.BlockSpec((1, HID), lambda i: (0, 0)),
        pl.BlockSpec((2 * HID, HID), lambda i: (0, 0)),
        pl.BlockSpec((1, HID), lambda i: (0, 0)),
    ],
    out_specs=pl.BlockSpec((RB, HID), lambda i: (i, 0)),
    out_shape=jax.ShapeDtypeStruct((R3, HID), jnp.float32),
)


def _tc_out_body(h0_ref, h1_ref, h2_ref, w_ref, b_ref, cz_ref, u_ref):
    w = w_ref[...]
    b = b_ref[...]
    cz_ref[...] = _dot(h0_ref[...], w) + b
    z1 = _dot(h1_ref[...], w) + b
    z2 = _dot(h2_ref[...], w) + b
    mean = (z1 + z2) * 0.5
    c1 = z1 - mean
    c2 = z2 - mean
    sq = (jnp.sum(c1 * c1, axis=1, keepdims=True)
          + jnp.sum(c2 * c2, axis=1, keepdims=True))
    u_ref[...] = jnp.maximum(sq, 1e-6)


_tc_out = pl.pallas_call(
    _tc_out_body,
    grid=(N // RB,),
    in_specs=[
        pl.BlockSpec((RB, HID), lambda i: (i, 0)),
        pl.BlockSpec((RB, HID), lambda i: (N // RB + i, 0)),
        pl.BlockSpec((RB, HID), lambda i: (2 * N // RB + i, 0)),
        pl.BlockSpec((HID, EMB), lambda i: (0, 0)),
        pl.BlockSpec((1, EMB), lambda i: (0, 0)),
    ],
    out_specs=[
        pl.BlockSpec((RB, EMB), lambda i: (i, 0)),
        pl.BlockSpec((RB, 1), lambda i: (i, 0)),
    ],
    out_shape=[
        jax.ShapeDtypeStruct((N, EMB), jnp.float32),
        jax.ShapeDtypeStruct((N, 1), jnp.float32),
    ],
)


def _tile_slabs(a, per_tile, chunks, pad_val):
    a = a.reshape(NW, per_tile)
    a = jnp.pad(a, ((0, 0), (0, chunks * B - per_tile)),
                constant_values=pad_val)
    return a.reshape(NW, chunks, B)


def kernel(x, edge_index, parent_mask, W_in, b_in, mW0, mb0, uW0, ub0,
           mW1, mb1, uW1, ub1, W_out, b_out):
    src = edge_index[0]
    dst = edge_index[1]
    m = parent_mask

    nk = jax.random.key(42)
    n0 = jax.random.normal(jax.random.fold_in(nk, 0), (N, IN_DIM), jnp.float32)
    n1 = jax.random.normal(jax.random.fold_in(nk, 1), (N, IN_DIM), jnp.float32)
    x3 = jnp.concatenate([x, x + 0.1 * n0, x + 0.1 * n1], axis=0)

    g_main = jnp.concatenate([src, src + N, src + 2 * N])
    g3d = _tile_slabs(g_main, 3 * E // NW, CH, R3)
    s_main = jnp.concatenate([jnp.where(m, dst, TRASHM),
                              jnp.where(m, dst + N, TRASHM),
                              jnp.where(m, dst + 2 * N, TRASHM)]).astype(jnp.int32)
    s3d = _tile_slabs(s_main, 3 * E // NW, CH, TRASHM)
    c1 = jnp.where(m, dst, TRASHC).astype(jnp.int32)
    c3d = _tile_slabs(c1, E // NW, CC, TRASHC)

    ones_b = jnp.ones((B, HID), jnp.float32)
    zm = jnp.zeros((ARM // 16, HID), jnp.float32)
    zc = jnp.zeros((ARC // 16, HID), jnp.float32)
    tail = jnp.concatenate([jnp.ones((1, HID), jnp.float32),
                            jnp.zeros((7, HID), jnp.float32)], axis=0)

    b_in2 = b_in.reshape(1, HID)
    mb0r = mb0.reshape(1, HID)
    ub0r = ub0.reshape(1, HID)
    mb1r = mb1.reshape(1, HID)
    ub1r = ub1.reshape(1, HID)
    b_out2 = b_out.reshape(1, EMB)

    h = _tc_in(x3, W_in, b_in2)
    pc = _sc_counts()(ones_b, c3d, zc)

    t0 = jnp.concatenate([h, tail], axis=0)
    p1 = _sc_main()(t0, g3d, s3d, zm)
    h = _tc_update(h, p1, pc, mW0, mb0r, uW0, ub0r)

    t1 = jnp.concatenate([h, tail], axis=0)
    p2 = _sc_main()(t1, g3d, s3d, zm)
    h = _tc_update(h, p2, pc, mW1, mb1r, uW1, ub1r)

    cz, u = _tc_out(h, h, h, W_out, b_out2)
    return cz, u[:, 0]
